# baseline (speedup 1.0000x reference)
import functools

import jax
import jax.numpy as jnp
from jax import lax
from jax.experimental import pallas as pl
from jax.experimental.pallas import tpu as pltpu
from jax.experimental.pallas import tpu_sc as plsc

N_NODES = 10000
N_EDGES = 320000
D = 128

NC = 2
NS = 16
NW = NC * NS
CH = 128
NCHUNK = 80
HALF = NCHUNK // 2
E_PAD = NW * NCHUNK * CH
N_PAD = 10240
ROWS_PT = N_PAD // NS
ZROWS = 128
LANES = 16


def _sc_segment_sum(feature, src3, dst3):
    mesh = plsc.VectorSubcoreMesh(core_axis_name="c", subcore_axis_name="s")

    @functools.partial(
        pl.kernel,
        mesh=mesh,
        out_type=jax.ShapeDtypeStruct((NC, N_PAD, D), jnp.float32),
        scratch_types=[
            pltpu.VMEM((HALF, CH), jnp.int32),
            pltpu.VMEM((HALF, CH), jnp.int32),
            pltpu.VMEM((CH, D), jnp.float32),
            pltpu.VMEM((CH, D), jnp.float32),
            pltpu.VMEM_SHARED((N_PAD, D), jnp.float32),
            pltpu.SemaphoreType.DMA,
            pltpu.SemaphoreType.DMA,
            pltpu.SemaphoreType.DMA,
            pltpu.SemaphoreType.DMA,
        ],
    )
    def k(feat_hbm, src_hbm, dst_hbm, out_hbm,
          sidx_v, didx_v, rows0, rows1, acc_sh, g0, g1, s0, s1):
        cid = lax.axis_index("c")
        sid = lax.axis_index("s")
        wid = sid * NC + cid

        zero = jnp.zeros((LANES,), jnp.float32)

        def zbody(i, carry):
            r = i // (D // LANES)
            col = (i % (D // LANES)) * LANES
            rows0[r, pl.ds(col, LANES)] = zero
            return carry

        lax.fori_loop(0, ZROWS * (D // LANES), zbody, 0)

        row0 = sid * ROWS_PT

        def zcopy(j, carry):
            pltpu.sync_copy(rows0, acc_sh.at[pl.ds(row0 + j * ZROWS, ZROWS)])
            return carry

        lax.fori_loop(0, ROWS_PT // ZROWS, zcopy, 0)
        plsc.subcore_barrier()

        def gather(c, buf, sem):
            return pltpu.async_copy(feat_hbm.at[sidx_v.at[c]], buf, sem)

        def gwait(buf, sem):
            pltpu.make_async_copy(feat_hbm.at[sidx_v.at[0]], buf, sem).wait()

        def scat(c, buf, sem):
            return pltpu.async_copy(buf, acc_sh.at[didx_v.at[c]], sem, add=True)

        def swait(buf, sem):
            pltpu.make_async_copy(buf, acc_sh.at[didx_v.at[0]], sem).wait()

        def epass(h, carry):
            pltpu.sync_copy(src_hbm.at[wid, pl.ds(h * HALF, HALF)], sidx_v)
            pltpu.sync_copy(dst_hbm.at[wid, pl.ds(h * HALF, HALF)], didx_v)
            gather(0, rows0, g0)

            def ebody(i, carry2):
                c0 = 2 * i
                c1 = c0 + 1
                c2 = jnp.where(c1 + 1 < HALF, c1 + 1, 0)
                gwait(rows0, g0)
                gather(c1, rows1, g1)
                pltpu.sync_copy(rows0, acc_sh.at[didx_v.at[c0]], add=True)
                gwait(rows1, g1)
                gather(c2, rows0, g0)
                pltpu.sync_copy(rows1, acc_sh.at[didx_v.at[c1]], add=True)
                return carry2

            lax.fori_loop(0, HALF // 2, ebody, 0)
            gwait(rows0, g0)
            return carry

        lax.fori_loop(0, 2, epass, 0)
        plsc.subcore_barrier()

        def wcopy(j, carry):
            r = row0 + j * ZROWS
            pltpu.sync_copy(acc_sh.at[pl.ds(r, ZROWS)], rows0)
            pltpu.sync_copy(rows0, out_hbm.at[cid, pl.ds(r, ZROWS)])
            return carry

        lax.fori_loop(0, ROWS_PT // ZROWS, wcopy, 0)

    return k(feature, src3, dst3)


def _tc_linear(partials, wt, bias):
    RB = 2048

    def mm(p_ref, w_ref, b_ref, o_ref):
        acc = p_ref[0] + p_ref[1]
        o_ref[...] = (
            jnp.dot(acc, w_ref[...], preferred_element_type=jnp.float32)
            + b_ref[...]
        )

    return pl.pallas_call(
        mm,
        grid=(N_PAD // RB,),
        in_specs=[
            pl.BlockSpec((NC, RB, D), lambda i: (0, i, 0)),
            pl.BlockSpec((D, D), lambda i: (0, 0)),
            pl.BlockSpec((1, D), lambda i: (0, 0)),
        ],
        out_specs=pl.BlockSpec((RB, D), lambda i: (i, 0)),
        out_shape=jax.ShapeDtypeStruct((N_PAD, D), jnp.float32),
    )(partials, wt, bias.reshape(1, D))


def kernel(feature, edge_index, W, b):
    ei = edge_index.astype(jnp.int32)
    npad = E_PAD - N_EDGES
    src3 = jnp.concatenate(
        [ei[0], jnp.zeros((npad,), jnp.int32)]).reshape(NW, NCHUNK, CH)
    pad_dst = N_NODES + jnp.arange(npad, dtype=jnp.int32) % (N_PAD - N_NODES)
    dst3 = jnp.concatenate([ei[1], pad_dst]).reshape(NW, NCHUNK, CH)
    partials = _sc_segment_sum(feature, src3, dst3)
    return _tc_linear(partials, W.T, b)[:N_NODES]


# speedup vs baseline: 3.0120x; 3.0120x over previous
import functools

import jax
import jax.numpy as jnp
from jax import lax
from jax.experimental import pallas as pl
from jax.experimental.pallas import tpu as pltpu
from jax.experimental.pallas import tpu_sc as plsc

N_NODES = 10000
N_EDGES = 320000
D = 128

NC = 2
NS = 16
NW = NC * NS
CH = 80
NCHUNK = 125
NPAIR = (NCHUNK - 1) // 2
N_PAD = 10240
ROWS_PT = N_PAD // NS
ZROWS = 80
LANES = 16


def _sc_segment_sum(feature, src3, dst3):
    mesh = plsc.VectorSubcoreMesh(core_axis_name="c", subcore_axis_name="s")

    @functools.partial(
        pl.kernel,
        mesh=mesh,
        out_type=jax.ShapeDtypeStruct((NC, N_PAD, D), jnp.float32),
        scratch_types=[
            pltpu.VMEM((NCHUNK * CH,), jnp.int32),
            pltpu.VMEM((NCHUNK, CH), jnp.int32),
            pltpu.VMEM((CH, D), jnp.float32),
            pltpu.VMEM((CH, D), jnp.float32),
            pltpu.VMEM_SHARED((N_PAD, D), jnp.float32),
            pltpu.SemaphoreType.DMA,
            pltpu.SemaphoreType.DMA,
        ],
    )
    def k(feat_hbm, src_hbm, dst_hbm, out_hbm,
          sidx_v, didx_v, rows0, rows1, acc_sh, g0, g1):
        cid = lax.axis_index("c")
        sid = lax.axis_index("s")
        wid = sid * NC + cid

        zero = jnp.zeros((LANES,), jnp.float32)

        def zbody(i, carry):
            r = i // (D // LANES)
            col = (i % (D // LANES)) * LANES
            rows0[r, pl.ds(col, LANES)] = zero
            return carry

        lax.fori_loop(0, ZROWS * (D // LANES), zbody, 0)

        row0 = sid * ROWS_PT

        def zcopy(j, carry):
            pltpu.sync_copy(rows0, acc_sh.at[pl.ds(row0 + j * ZROWS, ZROWS)])
            return carry

        lax.fori_loop(0, ROWS_PT // ZROWS, zcopy, 0)

        pltpu.sync_copy(src_hbm.at[wid], sidx_v)
        pltpu.sync_copy(dst_hbm.at[wid], didx_v)
        plsc.subcore_barrier()

        def gather(c, buf, sem):
            return pltpu.async_copy(
                feat_hbm.at[sidx_v.at[pl.ds(c * CH, CH)]], buf, sem)

        def gwait(buf, sem):
            pltpu.make_async_copy(
                feat_hbm.at[sidx_v.at[pl.ds(0, CH)]], buf, sem).wait()

        def scat(c, buf):
            pltpu.sync_copy(buf, acc_sh.at[didx_v.at[c]], add=True)

        gather(0, rows0, g0)

        def ebody(i, carry):
            c0 = 2 * i
            gwait(rows0, g0)
            gather(c0 + 1, rows1, g1)
            scat(c0, rows0)
            gwait(rows1, g1)
            gather(c0 + 2, rows0, g0)
            scat(c0 + 1, rows1)
            return carry

        lax.fori_loop(0, NPAIR, ebody, 0)
        gwait(rows0, g0)
        scat(NCHUNK - 1, rows0)
        plsc.subcore_barrier()

        def wcopy(j, carry):
            r = row0 + j * ZROWS
            pltpu.sync_copy(acc_sh.at[pl.ds(r, ZROWS)], rows0)
            pltpu.sync_copy(rows0, out_hbm.at[cid, pl.ds(r, ZROWS)])
            return carry

        lax.fori_loop(0, ROWS_PT // ZROWS, wcopy, 0)

    return k(feature, src3, dst3)


def _tc_linear(partials, wt, bias):
    RB = 2048

    def mm(p_ref, w_ref, b_ref, o_ref):
        acc = p_ref[0] + p_ref[1]
        o_ref[...] = (
            jnp.dot(acc, w_ref[...], preferred_element_type=jnp.float32)
            + b_ref[...]
        )

    return pl.pallas_call(
        mm,
        grid=(N_PAD // RB,),
        in_specs=[
            pl.BlockSpec((NC, RB, D), lambda i: (0, i, 0)),
            pl.BlockSpec((D, D), lambda i: (0, 0)),
            pl.BlockSpec((1, D), lambda i: (0, 0)),
        ],
        out_specs=pl.BlockSpec((RB, D), lambda i: (i, 0)),
        out_shape=jax.ShapeDtypeStruct((N_PAD, D), jnp.float32),
    )(partials, wt, bias.reshape(1, D))


def kernel(feature, edge_index, W, b):
    ei = edge_index.astype(jnp.int32)
    src3 = ei[0].reshape(NW, NCHUNK * CH)
    dst3 = ei[1].reshape(NW, NCHUNK, CH)
    partials = _sc_segment_sum(feature, src3, dst3)
    return _tc_linear(partials, W.T, b)[:N_NODES]


# speedup vs baseline: 3.1077x; 1.0318x over previous
import functools

import jax
import jax.numpy as jnp
from jax import lax
from jax.experimental import pallas as pl
from jax.experimental.pallas import tpu as pltpu
from jax.experimental.pallas import tpu_sc as plsc

N_NODES = 10000
N_EDGES = 320000
D = 128

NC = 2
NS = 16
NW = NC * NS
CH = 80
NCHUNK = 125
NPAIR = (NCHUNK - 1) // 2
N_PAD = 10240
ROWS_PT = N_PAD // NS
ZROWS = 80
LANES = 16


def _sc_segment_sum(feature, src3, dst3):
    mesh = plsc.VectorSubcoreMesh(core_axis_name="c", subcore_axis_name="s")

    @functools.partial(
        pl.kernel,
        mesh=mesh,
        out_type=jax.ShapeDtypeStruct((NC, N_PAD, D), jnp.float32),
        scratch_types=[
            pltpu.VMEM((NCHUNK * CH,), jnp.int32),
            pltpu.VMEM((NCHUNK, CH), jnp.int32),
            pltpu.VMEM((CH, D), jnp.float32),
            pltpu.VMEM((CH, D), jnp.float32),
            pltpu.VMEM_SHARED((N_PAD, D), jnp.float32),
            pltpu.SemaphoreType.DMA,
            pltpu.SemaphoreType.DMA,
            pltpu.SemaphoreType.DMA,
            pltpu.SemaphoreType.DMA,
        ],
    )
    def k(feat_hbm, src_hbm, dst_hbm, out_hbm,
          sidx_v, didx_v, rows0, rows1, acc_sh, g0, g1, s0, s1):
        cid = lax.axis_index("c")
        sid = lax.axis_index("s")
        wid = sid * NC + cid

        zero = jnp.zeros((LANES,), jnp.float32)

        def zbody(i, carry):
            r = i // (D // LANES)
            col = (i % (D // LANES)) * LANES
            rows0[r, pl.ds(col, LANES)] = zero
            return carry

        lax.fori_loop(0, ZROWS * (D // LANES), zbody, 0)

        row0 = sid * ROWS_PT

        def zcopy(j, carry):
            pltpu.sync_copy(rows0, acc_sh.at[pl.ds(row0 + j * ZROWS, ZROWS)])
            return carry

        lax.fori_loop(0, ROWS_PT // ZROWS, zcopy, 0)

        pltpu.sync_copy(src_hbm.at[wid], sidx_v)
        pltpu.sync_copy(dst_hbm.at[wid], didx_v)
        plsc.subcore_barrier()

        def gather(c, buf, sem):
            return pltpu.async_copy(
                feat_hbm.at[sidx_v.at[pl.ds(c * CH, CH)]], buf, sem)

        def gwait(buf, sem):
            pltpu.make_async_copy(
                feat_hbm.at[sidx_v.at[pl.ds(0, CH)]], buf, sem).wait()

        def scat(c, buf):
            pltpu.sync_copy(buf, acc_sh.at[didx_v.at[c]], add=True)

        gather(0, rows0, g0)

        def ebody(i, carry):
            c0 = 2 * i
            gwait(rows0, g0)
            gather(c0 + 1, rows1, g1)
            scat(c0, rows0)
            gwait(rows1, g1)
            gather(c0 + 2, rows0, g0)
            scat(c0 + 1, rows1)
            return carry

        lax.fori_loop(0, NPAIR, ebody, 0)
        gwait(rows0, g0)
        scat(NCHUNK - 1, rows0)
        plsc.subcore_barrier()

        for j in range(ROWS_PT // ZROWS):
            buf, sem = (rows0, s0) if j % 2 == 0 else (rows1, s1)
            r = row0 + j * ZROWS
            if j >= 2:
                pltpu.make_async_copy(buf, out_hbm.at[cid, pl.ds(r, ZROWS)],
                                      sem).wait()
            pltpu.sync_copy(acc_sh.at[pl.ds(r, ZROWS)], buf)
            pltpu.async_copy(buf, out_hbm.at[cid, pl.ds(r, ZROWS)], sem)
        for buf, sem in ((rows0, s0), (rows1, s1)):
            pltpu.make_async_copy(buf, out_hbm.at[cid, pl.ds(row0, ZROWS)],
                                  sem).wait()

    return k(feature, src3, dst3)


def _tc_linear(partials, wt, bias):
    RB = 2000

    def mm(p_ref, w_ref, b_ref, o_ref):
        acc = p_ref[0] + p_ref[1]
        o_ref[...] = (
            jnp.dot(acc, w_ref[...], preferred_element_type=jnp.float32)
            + b_ref[...]
        )

    return pl.pallas_call(
        mm,
        grid=(N_NODES // RB,),
        in_specs=[
            pl.BlockSpec((NC, RB, D), lambda i: (0, i, 0)),
            pl.BlockSpec((D, D), lambda i: (0, 0)),
            pl.BlockSpec((1, D), lambda i: (0, 0)),
        ],
        out_specs=pl.BlockSpec((RB, D), lambda i: (i, 0)),
        out_shape=jax.ShapeDtypeStruct((N_NODES, D), jnp.float32),
    )(partials, wt, bias.reshape(1, D))


def kernel(feature, edge_index, W, b):
    ei = edge_index.astype(jnp.int32)
    src3 = ei[0].reshape(NW, NCHUNK * CH)
    dst3 = ei[1].reshape(NW, NCHUNK, CH)
    partials = _sc_segment_sum(feature, src3, dst3)
    return _tc_linear(partials, W.T, b)


# speedup vs baseline: 4.1506x; 1.3356x over previous
import functools

import jax
import jax.numpy as jnp
from jax import lax
from jax.experimental import pallas as pl
from jax.experimental.pallas import tpu as pltpu
from jax.experimental.pallas import tpu_sc as plsc

N_NODES = 10000
N_EDGES = 320000
D = 128

NC = 2
NS = 16
NW = NC * NS
EPT = N_EDGES // NW
CH = 64
NCHUNK = EPT // CH
TAIL = EPT - NCHUNK * CH
N_PAD = 10240
ROWS_PT = N_PAD // NS
LANES = 16


def _sc_segment_sum(feature, src2, dst2):
    mesh = plsc.VectorSubcoreMesh(core_axis_name="c", subcore_axis_name="s")

    @functools.partial(
        pl.kernel,
        mesh=mesh,
        out_type=jax.ShapeDtypeStruct((NC, N_PAD, D), jnp.float32),
        scratch_types=[
            pltpu.VMEM((EPT,), jnp.int32),
            pltpu.VMEM((EPT,), jnp.int32),
            pltpu.VMEM((CH, D), jnp.float32),
            pltpu.VMEM((CH, D), jnp.float32),
            pltpu.VMEM((CH, D), jnp.float32),
            pltpu.VMEM_SHARED((N_PAD, D), jnp.float32),
            pltpu.SemaphoreType.DMA,
            pltpu.SemaphoreType.DMA,
            pltpu.SemaphoreType.DMA,
        ],
    )
    def k(feat_hbm, src_hbm, dst_hbm, out_hbm,
          sidx_v, didx_v, rows0, rows1, rows2, acc_sh, g0, g1, g2):
        cid = lax.axis_index("c")
        sid = lax.axis_index("s")
        wid = sid * NC + cid

        zero = jnp.zeros((LANES,), jnp.float32)

        def zbody(i, carry):
            r = i // (D // LANES)
            col = (i % (D // LANES)) * LANES
            rows0[r, pl.ds(col, LANES)] = zero
            return carry

        lax.fori_loop(0, CH * (D // LANES), zbody, 0)

        row0 = sid * ROWS_PT

        def zcopy(j, carry):
            pltpu.sync_copy(rows0, acc_sh.at[pl.ds(row0 + j * CH, CH)])
            return carry

        lax.fori_loop(0, ROWS_PT // CH, zcopy, 0)

        pltpu.sync_copy(src_hbm.at[wid], sidx_v)
        pltpu.sync_copy(dst_hbm.at[wid], didx_v)
        plsc.subcore_barrier()

        def gather(c, buf, sem):
            pltpu.async_copy(
                feat_hbm.at[sidx_v.at[pl.ds(c * CH, CH)]], buf, sem)

        def gwait(buf, sem):
            pltpu.make_async_copy(
                feat_hbm.at[sidx_v.at[pl.ds(0, CH)]], buf, sem).wait()

        def scat(c, buf):
            pltpu.sync_copy(
                buf, acc_sh.at[didx_v.at[pl.ds(c * CH, CH)]], add=True)

        bufs = ((rows0, g0), (rows1, g1), (rows2, g2))
        gather(0, rows0, g0)
        gather(1, rows1, g1)

        def ebody(i, carry):
            c = 3 * i
            for t in range(3):
                buf, sem = bufs[t]
                nbuf, nsem = bufs[(t + 2) % 3]
                cn = jnp.where(c + t + 2 < NCHUNK, c + t + 2, 0)
                gwait(buf, sem)
                gather(cn, nbuf, nsem)
                scat(c + t, buf)
            return carry

        lax.fori_loop(0, NCHUNK // 3, ebody, 0)
        gwait(rows0, g0)
        gwait(rows1, g1)
        pltpu.async_copy(
            feat_hbm.at[sidx_v.at[pl.ds(NCHUNK * CH, TAIL)]],
            rows0.at[pl.ds(0, TAIL)], g0).wait()
        pltpu.sync_copy(
            rows0.at[pl.ds(0, TAIL)],
            acc_sh.at[didx_v.at[pl.ds(NCHUNK * CH, TAIL)]], add=True)
        plsc.subcore_barrier()

        for j in range(ROWS_PT // CH):
            buf, sem = bufs[j % 3]
            r = row0 + j * CH
            if j >= 3:
                pltpu.make_async_copy(buf, out_hbm.at[cid, pl.ds(r, CH)],
                                      sem).wait()
            pltpu.sync_copy(acc_sh.at[pl.ds(r, CH)], buf)
            pltpu.async_copy(buf, out_hbm.at[cid, pl.ds(r, CH)], sem)
        for buf, sem in bufs:
            pltpu.make_async_copy(buf, out_hbm.at[cid, pl.ds(row0, CH)],
                                  sem).wait()

    return k(feature, src2, dst2)


def _tc_linear(partials, wt, bias):
    RB = 2000

    def mm(p_ref, w_ref, b_ref, o_ref):
        acc = p_ref[0] + p_ref[1]
        o_ref[...] = (
            jnp.dot(acc, w_ref[...], preferred_element_type=jnp.float32)
            + b_ref[...]
        )

    return pl.pallas_call(
        mm,
        grid=(N_NODES // RB,),
        in_specs=[
            pl.BlockSpec((NC, RB, D), lambda i: (0, i, 0)),
            pl.BlockSpec((D, D), lambda i: (0, 0)),
            pl.BlockSpec((1, D), lambda i: (0, 0)),
        ],
        out_specs=pl.BlockSpec((RB, D), lambda i: (i, 0)),
        out_shape=jax.ShapeDtypeStruct((N_NODES, D), jnp.float32),
    )(partials, wt, bias.reshape(1, D))


def kernel(feature, edge_index, W, b):
    ei = edge_index.astype(jnp.int32)
    src2 = ei[0].reshape(NW, EPT)
    dst2 = ei[1].reshape(NW, EPT)
    partials = _sc_segment_sum(feature, src2, dst2)
    return _tc_linear(partials, W.T, b)


# speedup vs baseline: 4.5825x; 1.1041x over previous
import functools

import jax
import jax.numpy as jnp
from jax import lax
from jax.experimental import pallas as pl
from jax.experimental.pallas import tpu as pltpu
from jax.experimental.pallas import tpu_sc as plsc

N_NODES = 10000
N_EDGES = 320000
D = 128

NC = 2
NS = 16
NW = NC * NS
EPT = N_EDGES // NW
CH = 40
NCHUNK = EPT // CH
NBUF = 5
N_PAD = 10240
ROWS_PT = N_PAD // NS
LANES = 16


def _sc_segment_sum(feature, src2, dst2):
    mesh = plsc.VectorSubcoreMesh(core_axis_name="c", subcore_axis_name="s")

    @functools.partial(
        pl.kernel,
        mesh=mesh,
        out_type=jax.ShapeDtypeStruct((NC, N_PAD, D), jnp.float32),
        scratch_types=[
            pltpu.VMEM((EPT,), jnp.int32),
            pltpu.VMEM((EPT,), jnp.int32),
        ] + [pltpu.VMEM((CH, D), jnp.float32)] * NBUF
          + [pltpu.VMEM_SHARED((N_PAD, D), jnp.float32)]
          + [pltpu.SemaphoreType.DMA] * NBUF,
    )
    def k(feat_hbm, src_hbm, dst_hbm, out_hbm, sidx_v, didx_v, *rest):
        rbufs = rest[:NBUF]
        acc_sh = rest[NBUF]
        sems = rest[NBUF + 1:]
        rows0 = rbufs[0]
        cid = lax.axis_index("c")
        sid = lax.axis_index("s")
        wid = sid * NC + cid

        zero = jnp.zeros((LANES,), jnp.float32)

        def zbody(i, carry):
            r = i // (D // LANES)
            col = (i % (D // LANES)) * LANES
            rows0[r, pl.ds(col, LANES)] = zero
            return carry

        lax.fori_loop(0, CH * (D // LANES), zbody, 0)

        row0 = sid * ROWS_PT

        def zcopy(j, carry):
            pltpu.sync_copy(rows0, acc_sh.at[pl.ds(row0 + j * CH, CH)])
            return carry

        lax.fori_loop(0, ROWS_PT // CH, zcopy, 0)

        pltpu.sync_copy(src_hbm.at[wid], sidx_v)
        pltpu.sync_copy(dst_hbm.at[wid], didx_v)
        plsc.subcore_barrier()

        def gather(c, buf, sem):
            pltpu.async_copy(
                feat_hbm.at[sidx_v.at[pl.ds(c * CH, CH)]], buf, sem)

        def gwait(buf, sem):
            pltpu.make_async_copy(
                feat_hbm.at[sidx_v.at[pl.ds(0, CH)]], buf, sem).wait()

        def scat(c, buf):
            pltpu.sync_copy(
                buf, acc_sh.at[didx_v.at[pl.ds(c * CH, CH)]], add=True)

        bufs = tuple(zip(rbufs, sems))
        for t in range(NBUF - 1):
            gather(t, rbufs[t], sems[t])

        def ebody(i, carry):
            c = NBUF * i
            for t in range(NBUF):
                buf, sem = bufs[t]
                nbuf, nsem = bufs[(t + NBUF - 1) % NBUF]
                cn = jnp.where(c + t + NBUF - 1 < NCHUNK, c + t + NBUF - 1, 0)
                gwait(buf, sem)
                gather(cn, nbuf, nsem)
                scat(c + t, buf)
            return carry

        lax.fori_loop(0, NCHUNK // NBUF, ebody, 0)
        for t in range(NBUF - 1):
            gwait(rbufs[t], sems[t])
        plsc.subcore_barrier()

        for j in range(ROWS_PT // CH):
            buf, sem = bufs[j % NBUF]
            r = row0 + j * CH
            if j >= NBUF:
                pltpu.make_async_copy(buf, out_hbm.at[cid, pl.ds(r, CH)],
                                      sem).wait()
            pltpu.sync_copy(acc_sh.at[pl.ds(r, CH)], buf)
            pltpu.async_copy(buf, out_hbm.at[cid, pl.ds(r, CH)], sem)
        for buf, sem in bufs:
            pltpu.make_async_copy(buf, out_hbm.at[cid, pl.ds(row0, CH)],
                                  sem).wait()

    return k(feature, src2, dst2)


def _tc_linear(partials, wt, bias):
    RB = 2000

    def mm(p_ref, w_ref, b_ref, o_ref):
        acc = p_ref[0] + p_ref[1]
        o_ref[...] = (
            jnp.dot(acc, w_ref[...], preferred_element_type=jnp.float32)
            + b_ref[...]
        )

    return pl.pallas_call(
        mm,
        grid=(N_NODES // RB,),
        in_specs=[
            pl.BlockSpec((NC, RB, D), lambda i: (0, i, 0)),
            pl.BlockSpec((D, D), lambda i: (0, 0)),
            pl.BlockSpec((1, D), lambda i: (0, 0)),
        ],
        out_specs=pl.BlockSpec((RB, D), lambda i: (i, 0)),
        out_shape=jax.ShapeDtypeStruct((N_NODES, D), jnp.float32),
    )(partials, wt, bias.reshape(1, D))


def kernel(feature, edge_index, W, b):
    ei = edge_index.astype(jnp.int32)
    src2 = ei[0].reshape(NW, EPT)
    dst2 = ei[1].reshape(NW, EPT)
    partials = _sc_segment_sum(feature, src2, dst2)
    return _tc_linear(partials, W.T, b)
